# baseline (speedup 1.0000x reference)
import functools

import jax
import jax.numpy as jnp
from jax import lax
from jax.experimental import pallas as pl
from jax.experimental.pallas import tpu as pltpu
from jax.experimental.pallas import tpu_sc as plsc

N = 10000
E = 320000
D = 128

NC = 2
NS = 16
NW = NC * NS
CHUNK = 48
RING = 4
PD = 2
NCHUNKS = 216
EPW = NCHUNKS * CHUNK
E_PAD = NW * EPW
ROWS_PER_S = N // NS
OUTER = NCHUNKS // RING


def _spmm_body(h_hbm, row_hbm, col_hbm, val_hbm, out_hbm,
               acc, colv, valv, rowv, gbuf, gsem, rsem, ssem):
    cid = lax.axis_index("c")
    sid = lax.axis_index("s")
    w = cid * NS + sid
    ebase = w * EPW

    pltpu.sync_copy(col_hbm.at[pl.ds(ebase, EPW)], colv)
    pltpu.sync_copy(val_hbm.at[pl.ds(ebase, EPW)], valv)

    def zero_body(e, _):
        for j in range(D // 16):
            gbuf[0, e, pl.ds(j * 16, 16)] = jnp.zeros((16,), jnp.float32)
        return 0

    lax.fori_loop(0, CHUNK, zero_body, 0)
    abase = sid * ROWS_PER_S
    for k in range(ROWS_PER_S // CHUNK):
        pltpu.sync_copy(gbuf.at[0], acc.at[pl.ds(abase + k * CHUNK, CHUNK)])
    rem = ROWS_PER_S % CHUNK
    pltpu.sync_copy(gbuf.at[0, pl.ds(0, rem)],
                    acc.at[pl.ds(abase + ROWS_PER_S - rem, rem)])

    for c in range(PD):
        pltpu.async_copy(row_hbm.at[pl.ds(ebase + c * CHUNK, CHUNK)],
                         rowv.at[c], rsem.at[c])
        pltpu.async_copy(h_hbm.at[colv.at[pl.ds(c * CHUNK, CHUNK)]],
                         gbuf.at[c], gsem.at[c])

    plsc.subcore_barrier()

    def outer_body(o, _):
        for b in range(RING):
            s = o * RING + b
            pltpu.make_async_copy(h_hbm.at[pl.ds(0, CHUNK)],
                                  gbuf.at[b], gsem.at[b]).wait()

            def scale_body(g, _, b=b, s=s):
                vv = valv[pl.ds(s * CHUNK + g * 16, 16)]
                for l in range(16):
                    v = vv[l]
                    for j in range(D // 16):
                        gbuf[b, g * 16 + l, pl.ds(j * 16, 16)] = (
                            gbuf[b, g * 16 + l, pl.ds(j * 16, 16)] * v)
                return 0

            lax.fori_loop(0, CHUNK // 16, scale_body, 0)

            pltpu.make_async_copy(row_hbm.at[pl.ds(0, CHUNK)],
                                  rowv.at[b], rsem.at[b]).wait()
            pltpu.async_copy(gbuf.at[b], acc.at[rowv.at[b]], ssem.at[b],
                             add=True)

            nb = (b + PD) % RING

            def issue(o=o, b=b, nb=nb):
                t = o * RING + b + PD
                pltpu.async_copy(row_hbm.at[pl.ds(ebase + t * CHUNK, CHUNK)],
                                 rowv.at[nb], rsem.at[nb])
                pltpu.async_copy(h_hbm.at[colv.at[pl.ds(t * CHUNK, CHUNK)]],
                                 gbuf.at[nb], gsem.at[nb])

            def drain(nb=nb):
                pltpu.make_async_copy(h_hbm.at[pl.ds(0, CHUNK)],
                                      gbuf.at[nb], ssem.at[nb]).wait()

            if b < PD:
                @pl.when(o > 0)
                def _(issue=issue, drain=drain):
                    drain()
                    issue()

                @pl.when(o == 0)
                def _(issue=issue):
                    issue()
            else:
                @pl.when(o < OUTER - 1)
                def _(issue=issue, drain=drain):
                    drain()
                    issue()
        return 0

    lax.fori_loop(0, OUTER, outer_body, 0)

    for b in range(RING):
        pltpu.make_async_copy(h_hbm.at[pl.ds(0, CHUNK)],
                              gbuf.at[b], ssem.at[b]).wait()

    plsc.subcore_barrier()

    off = pl.multiple_of(sid * 624, 8)
    pltpu.sync_copy(acc.at[pl.ds(off, 624)], out_hbm.at[cid, pl.ds(off, 624)])

    @pl.when(sid == 0)
    def _():
        pltpu.sync_copy(acc.at[pl.ds(NS * 624, N - NS * 624)],
                        out_hbm.at[cid, pl.ds(NS * 624, N - NS * 624)])


@jax.jit
def _spmm_sc(h, row, col, vals):
    mesh = plsc.VectorSubcoreMesh(core_axis_name="c", subcore_axis_name="s")
    return pl.kernel(
        _spmm_body,
        mesh=mesh,
        out_type=jax.ShapeDtypeStruct((NC, N, D), jnp.float32),
        scratch_types=[
            pltpu.VMEM_SHARED((N, D), jnp.float32),
            pltpu.VMEM((EPW,), jnp.int32),
            pltpu.VMEM((EPW,), jnp.float32),
            pltpu.VMEM((RING, CHUNK), jnp.int32),
            pltpu.VMEM((RING, CHUNK, D), jnp.float32),
            pltpu.SemaphoreType.DMA((RING,)),
            pltpu.SemaphoreType.DMA((RING,)),
            pltpu.SemaphoreType.DMA((RING,)),
        ],
    )(h, row, col, vals)


def _dense_body(h_ref, w_ref, b_ref, o_ref, *, act, sum2):
    h = h_ref[...]
    if sum2:
        h = h[0] + h[1]
    if act:
        h = jnp.maximum(h, 0.0)
    o_ref[...] = (jnp.dot(h, w_ref[...], preferred_element_type=jnp.float32)
                  + b_ref[...])


def _dense_tc(h, w, b, act, sum2):
    n = h.shape[-2]
    return pl.pallas_call(
        functools.partial(_dense_body, act=act, sum2=sum2),
        out_shape=jax.ShapeDtypeStruct((n, w.shape[1]), jnp.float32),
    )(h, w, b.reshape(1, -1))


def kernel(x, edge_index, adj_values, W1, b1, W2, b2, Wc, bc):
    pad = E_PAD - E
    row = jnp.concatenate(
        [edge_index[0].astype(jnp.int32), jnp.zeros((pad,), jnp.int32)])
    col = jnp.concatenate(
        [edge_index[1].astype(jnp.int32), jnp.zeros((pad,), jnp.int32)])
    vals = jnp.concatenate(
        [adj_values.astype(jnp.float32), jnp.zeros((pad,), jnp.float32)])
    h = _dense_tc(x, W1, b1, act=False, sum2=False)
    p = _spmm_sc(h, row, col, vals)
    h2 = _dense_tc(p, W2, b2, act=True, sum2=True)
    q = _spmm_sc(h2, row, col, vals)
    return _dense_tc(q, Wc, bc, act=False, sum2=True)


# speedup vs baseline: 4.1210x; 4.1210x over previous
import functools

import jax
import jax.numpy as jnp
from jax import lax
from jax.experimental import pallas as pl
from jax.experimental.pallas import tpu as pltpu
from jax.experimental.pallas import tpu_sc as plsc

N = 10000
E = 320000
D = 128

NC = 2
NS = 16
NW = NC * NS
CHUNK = 48
RING = 4
PD = 2
NCHUNKS = 216
EPW = NCHUNKS * CHUNK
E_PAD = NW * EPW
ROWS_PER_S = N // NS
OUTER = NCHUNKS // RING


def _spmm_body(h_hbm, row_hbm, col_hbm, val_hbm, out_hbm,
               acc, colv, valv, rowv, gbuf, gsem, rsem, ssem):
    cid = lax.axis_index("c")
    sid = lax.axis_index("s")
    w = cid * NS + sid
    ebase = w * EPW

    pltpu.sync_copy(col_hbm.at[pl.ds(ebase, EPW)], colv)
    pltpu.sync_copy(val_hbm.at[pl.ds(ebase, EPW)], valv)

    def zero_body(e, _):
        for j in range(D // 16):
            gbuf[0, e, pl.ds(j * 16, 16)] = jnp.zeros((16,), jnp.float32)
        return 0

    lax.fori_loop(0, CHUNK, zero_body, 0)
    abase = sid * ROWS_PER_S
    for k in range(ROWS_PER_S // CHUNK):
        pltpu.sync_copy(gbuf.at[0], acc.at[pl.ds(abase + k * CHUNK, CHUNK)])
    rem = ROWS_PER_S % CHUNK
    pltpu.sync_copy(gbuf.at[0, pl.ds(0, rem)],
                    acc.at[pl.ds(abase + ROWS_PER_S - rem, rem)])

    for c in range(PD):
        pltpu.async_copy(row_hbm.at[pl.ds(ebase + c * CHUNK, CHUNK)],
                         rowv.at[c], rsem.at[c])
        pltpu.async_copy(h_hbm.at[colv.at[pl.ds(c * CHUNK, CHUNK)]],
                         gbuf.at[c], gsem.at[c])

    plsc.subcore_barrier()

    def outer_body(o, _):
        for b in range(RING):
            s = o * RING + b
            pltpu.make_async_copy(h_hbm.at[pl.ds(0, CHUNK)],
                                  gbuf.at[b], gsem.at[b]).wait()

            def scale_body(g, _, b=b, s=s):
                vv = valv[pl.ds(s * CHUNK + g * 16, 16)]
                for l in range(16):
                    v = vv[l]
                    for j in range(D // 16):
                        gbuf[b, g * 16 + l, pl.ds(j * 16, 16)] = (
                            gbuf[b, g * 16 + l, pl.ds(j * 16, 16)] * v)
                return 0

            lax.fori_loop(0, CHUNK // 16, scale_body, 0)

            pltpu.make_async_copy(row_hbm.at[pl.ds(0, CHUNK)],
                                  rowv.at[b], rsem.at[b]).wait()
            pltpu.async_copy(gbuf.at[b], acc.at[rowv.at[b]], ssem.at[b],
                             add=True)

            nb = (b + PD) % RING

            def issue(o=o, b=b, nb=nb):
                t = o * RING + b + PD
                pltpu.async_copy(row_hbm.at[pl.ds(ebase + t * CHUNK, CHUNK)],
                                 rowv.at[nb], rsem.at[nb])
                pltpu.async_copy(h_hbm.at[colv.at[pl.ds(t * CHUNK, CHUNK)]],
                                 gbuf.at[nb], gsem.at[nb])

            def drain(nb=nb):
                pltpu.make_async_copy(h_hbm.at[pl.ds(0, CHUNK)],
                                      gbuf.at[nb], ssem.at[nb]).wait()

            if b < PD:
                @pl.when(o > 0)
                def _(issue=issue, drain=drain):
                    drain()
                    issue()

                @pl.when(o == 0)
                def _(issue=issue):
                    issue()
            else:
                @pl.when(o < OUTER - 1)
                def _(issue=issue, drain=drain):
                    drain()
                    issue()
        return 0

    lax.fori_loop(0, OUTER, outer_body, 0)

    for b in range(RING):
        pltpu.make_async_copy(h_hbm.at[pl.ds(0, CHUNK)],
                              gbuf.at[b], ssem.at[b]).wait()

    plsc.subcore_barrier()

    off = pl.multiple_of(sid * 624, 8)
    pltpu.sync_copy(acc.at[pl.ds(off, 624)], out_hbm.at[cid, pl.ds(off, 624)])

    @pl.when(sid == 0)
    def _():
        pltpu.sync_copy(acc.at[pl.ds(NS * 624, N - NS * 624)],
                        out_hbm.at[cid, pl.ds(NS * 624, N - NS * 624)])


@jax.jit
def _spmm_sc(h, row, col, vals):
    mesh = plsc.VectorSubcoreMesh(core_axis_name="c", subcore_axis_name="s")
    return pl.kernel(
        _spmm_body,
        mesh=mesh,
        out_type=jax.ShapeDtypeStruct((NC, N, D), jnp.float32),
        scratch_types=[
            pltpu.VMEM_SHARED((N, D), jnp.float32),
            pltpu.VMEM((EPW,), jnp.int32),
            pltpu.VMEM((EPW,), jnp.float32),
            pltpu.VMEM((RING, CHUNK), jnp.int32),
            pltpu.VMEM((RING, CHUNK, D), jnp.float32),
            pltpu.SemaphoreType.DMA((RING,)),
            pltpu.SemaphoreType.DMA((RING,)),
            pltpu.SemaphoreType.DMA((RING,)),
        ],
    )(h, row, col, vals)


def _dense_body(h_ref, w_ref, b_ref, o_ref, *, act, sum2):
    h = h_ref[...]
    if sum2:
        h = h[0] + h[1]
    if act:
        h = jnp.maximum(h, 0.0)
    o_ref[...] = (jnp.dot(h, w_ref[...], preferred_element_type=jnp.float32)
                  + b_ref[...])


def _dense_tc(h, w, b, act, sum2):
    n = h.shape[-2]
    return pl.pallas_call(
        functools.partial(_dense_body, act=act, sum2=sum2),
        out_shape=jax.ShapeDtypeStruct((n, w.shape[1]), jnp.float32),
    )(h, w, b.reshape(1, -1))


def kernel(x, edge_index, adj_values, W1, b1, W2, b2, Wc, bc):
    pad = E_PAD - E
    spread = (jnp.arange(pad, dtype=jnp.int32) * 13) % N
    row = jnp.concatenate([edge_index[0].astype(jnp.int32), spread])
    col = jnp.concatenate([edge_index[1].astype(jnp.int32), spread])
    vals = jnp.concatenate(
        [adj_values.astype(jnp.float32), jnp.zeros((pad,), jnp.float32)])
    h = _dense_tc(x, W1, b1, act=False, sum2=False)
    p = _spmm_sc(h, row, col, vals)
    h2 = _dense_tc(p, W2, b2, act=True, sum2=True)
    q = _spmm_sc(h2, row, col, vals)
    return _dense_tc(q, Wc, bc, act=False, sum2=True)
